# baseline (speedup 1.0000x reference)
import functools

import jax
import jax.numpy as jnp
from jax.experimental import pallas as pl
from jax.experimental.pallas import tpu as pltpu

N = 10000
E = 320000
FS = 64
EPS = 1e-5
TE = 4000


def _mlp1_body(xi_ref, d_ref, a_ref, c_ref, w_ref, o_ref):
    din = xi_ref.shape[1]
    a = a_ref[0, :]
    c = c_ref[0, :]
    hi = jnp.maximum(xi_ref[...] * a[:din] + c[:din], 0.0)
    hd = jnp.maximum(d_ref[...] * a[din:] + c[din:], 0.0)
    h = jnp.concatenate([hi, hd], axis=1)
    o_ref[...] = jnp.dot(h, w_ref[...], preferred_element_type=jnp.float32,
                         precision=jax.lax.Precision.DEFAULT)


def _mlp2_body(x1_ref, a_ref, c_ref, w_ref, o_ref):
    h = jnp.maximum(x1_ref[...] * a_ref[0, :] + c_ref[0, :], 0.0)
    o_ref[...] = jnp.dot(h, w_ref[...], preferred_element_type=jnp.float32,
                         precision=jax.lax.Precision.DEFAULT)


def _mlp1(xi, d, a, c, w):
    din = xi.shape[1]
    grid = (E // TE,)
    return pl.pallas_call(
        _mlp1_body,
        grid=grid,
        in_specs=[
            pl.BlockSpec((TE, din), lambda i: (i, 0)),
            pl.BlockSpec((TE, din), lambda i: (i, 0)),
            pl.BlockSpec((1, 2 * din), lambda i: (0, 0)),
            pl.BlockSpec((1, 2 * din), lambda i: (0, 0)),
            pl.BlockSpec((2 * din, FS), lambda i: (0, 0)),
        ],
        out_specs=pl.BlockSpec((TE, FS), lambda i: (i, 0)),
        out_shape=jax.ShapeDtypeStruct((E, FS), jnp.float32),
    )(xi, d, a.reshape(1, -1), c.reshape(1, -1), w)


def _mlp2(x1, a, c, w):
    grid = (E // TE,)
    return pl.pallas_call(
        _mlp2_body,
        grid=grid,
        in_specs=[
            pl.BlockSpec((TE, FS), lambda i: (i, 0)),
            pl.BlockSpec((1, FS), lambda i: (0, 0)),
            pl.BlockSpec((1, FS), lambda i: (0, 0)),
            pl.BlockSpec((FS, FS), lambda i: (0, 0)),
        ],
        out_specs=pl.BlockSpec((TE, FS), lambda i: (i, 0)),
        out_shape=jax.ShapeDtypeStruct((E, FS), jnp.float32),
    )(x1, a.reshape(1, -1), c.reshape(1, -1), w)


def _bn_coeffs(m, v, g, b):
    inv = g / jnp.sqrt(v + EPS)
    return inv, b - m * inv


def _edge_conv(x, edge_index, p):
    g1, b1, W1, g2, b2, W2 = p
    src = edge_index[0]
    dst = edge_index[1]
    xi = x[dst]
    d = x[src] - xi
    m1 = jnp.concatenate([jnp.mean(xi, 0), jnp.mean(d, 0)])
    v1 = jnp.concatenate([jnp.var(xi, 0), jnp.var(d, 0)])
    a1, c1 = _bn_coeffs(m1, v1, g1, b1)
    x1 = _mlp1(xi, d, a1, c1, W1)
    a2, c2 = _bn_coeffs(jnp.mean(x1, 0), jnp.var(x1, 0), g2, b2)
    x2 = _mlp2(x1, a2, c2, W2)
    out = jax.ops.segment_max(x2, dst, num_segments=N)
    return jnp.where(jnp.isfinite(out), out, 0.0)


def kernel(x, spatial_edge_index, temporal_edge_index, params, fcW, fcb):
    g1s = _edge_conv(x, spatial_edge_index, params[0])
    g1st = _edge_conv(g1s, temporal_edge_index, params[4])
    g2s = _edge_conv(g1st, spatial_edge_index, params[1])
    g2st = _edge_conv(g2s, temporal_edge_index, params[5]) + g1st
    g3s = _edge_conv(g2st, spatial_edge_index, params[2])
    g3st = _edge_conv(g3s, temporal_edge_index, params[6]) + g2st
    g4s = _edge_conv(g3st, spatial_edge_index, params[3])
    g4st = _edge_conv(g4s, temporal_edge_index, params[7]) + g3st
    return jnp.dot(g4st, fcW) + fcb
